# baseline (speedup 1.0000x reference)
import functools

import jax
import jax.numpy as jnp
from jax import lax
from jax.experimental import pallas as pl
from jax.experimental.pallas import tpu as pltpu
from jax.experimental.pallas import tpu_sc as plsc

HIDDEN = 64
PADW = 128
SEQ_LEN = 200
BATCH = 4096
NC, NS = 2, 16
NW = NC * NS
BPW = BATCH // NW
SPLIT = 104
LANES = 16


def _build():
    mesh = plsc.VectorSubcoreMesh(core_axis_name="c", subcore_axis_name="s")

    @functools.partial(
        pl.kernel,
        out_type=jax.ShapeDtypeStruct((BATCH, SEQ_LEN, PADW), jnp.float32),
        mesh=mesh,
        scratch_types=[
            pltpu.VMEM((BPW, SEQ_LEN), jnp.int32),
            pltpu.VMEM((2, SEQ_LEN, HIDDEN), jnp.float32),
            pltpu.VMEM((2, SEQ_LEN, PADW), jnp.float32),
            pltpu.VMEM((SEQ_LEN, HIDDEN), jnp.float32),
            [pltpu.SemaphoreType.DMA] * 2,
            [pltpu.SemaphoreType.DMA] * 2,
        ],
        compiler_params=pltpu.CompilerParams(use_tc_tiling_on_sc=False),
    )
    def k(idx_hbm, table_hbm, pos_hbm, out_hbm, idx_v, gbufs, obufs, pos_v,
          gsems, ssems):
        wid = lax.axis_index("s") * NC + lax.axis_index("c")
        pltpu.sync_copy(pos_hbm, pos_v)
        pltpu.sync_copy(idx_hbm.at[pl.ds(wid * BPW, BPW)], idx_v)

        def issue_gather(i, k_static):
            gbuf = gbufs.at[k_static]
            pltpu.async_copy(table_hbm.at[idx_v.at[i, pl.ds(0, SPLIT)]],
                             gbuf.at[pl.ds(0, SPLIT)], gsems[k_static])
            pltpu.async_copy(
                table_hbm.at[idx_v.at[i, pl.ds(SPLIT, SEQ_LEN - SPLIT)]],
                gbuf.at[pl.ds(SPLIT, SEQ_LEN - SPLIT)], gsems[k_static])

        def wait_gather(k_static):
            gbuf = gbufs.at[k_static]
            pltpu.make_async_copy(table_hbm.at[pl.ds(0, SPLIT)],
                                  gbuf.at[pl.ds(0, SPLIT)],
                                  gsems[k_static]).wait()
            pltpu.make_async_copy(table_hbm.at[pl.ds(0, SEQ_LEN - SPLIT)],
                                  gbuf.at[pl.ds(SPLIT, SEQ_LEN - SPLIT)],
                                  gsems[k_static]).wait()

        def wait_store(k_static):
            pltpu.make_async_copy(obufs.at[k_static], out_hbm.at[0],
                                  ssems[k_static]).wait()

        issue_gather(0, 0)
        issue_gather(1, 1)

        @pl.loop(0, BPW // 2)
        def _grp(j):
            for kk in range(2):
                i = j * 2 + kk
                gbuf = gbufs.at[kk]
                obuf = obufs.at[kk]
                wait_gather(kk)

                @pl.when(j > 0)
                def _():
                    wait_store(kk)

                @pl.loop(0, SEQ_LEN)
                def _row(r):
                    for d in range(HIDDEN // LANES):
                        sl = pl.ds(d * LANES, LANES)
                        obuf[r, sl] = gbuf[r, sl] + pos_v[r, sl]

                pltpu.async_copy(obuf, out_hbm.at[wid * BPW + i], ssems[kk])

                @pl.when(j < BPW // 2 - 1)
                def _():
                    issue_gather(i + 2, kk)

        wait_store(0)
        wait_store(1)

    return k


_KERNEL = _build()


def kernel(item_seq, ID_embeddings, positional_embeddings):
    out = _KERNEL(item_seq, ID_embeddings, positional_embeddings)
    return out[:, :, :HIDDEN]
